# baseline (speedup 1.0000x reference)
import functools

import jax
import jax.numpy as jnp
from jax import lax
from jax.experimental import pallas as pl
from jax.experimental.pallas import tpu as pltpu
from jax.experimental.pallas import tpu_sc as plsc

_NC = 2
_NS = 16
_NW = _NC * _NS
_T = 16384


def _sc_body(x_hbm, pe_hbm, o_hbm, peb, xb, lsem, ssem):
    LD = pe_hbm.shape[0]
    Bn = x_hbm.shape[0] // LD
    ch = LD // _NW
    nt = ch // _T
    wid = lax.axis_index("s") * _NC + lax.axis_index("c")
    base = wid * ch

    def x_load(k, slot):
        t = k // Bn
        b = k % Bn
        off = b * LD + base + t * _T
        pltpu.async_copy(x_hbm.at[pl.ds(off, _T)], xb.at[slot], lsem.at[slot])

    x_load(0, 0)

    def t_body(t, _):
        pltpu.sync_copy(pe_hbm.at[pl.ds(base + t * _T, _T)], peb)
        for b in range(Bn):
            s = b & 1
            k = t * Bn + b
            pltpu.make_async_copy(
                x_hbm.at[pl.ds(0, _T)], xb.at[s], lsem.at[s]).wait()
            if b == Bn - 1:
                @pl.when(t + 1 < nt)
                def _():
                    pltpu.make_async_copy(
                        xb.at[1 - s], o_hbm.at[pl.ds(0, _T)],
                        ssem.at[1 - s]).wait()
                    x_load(k + 1, 1 - s)
            elif b == 0:
                @pl.when(t >= 1)
                def _():
                    pltpu.make_async_copy(
                        xb.at[1 - s], o_hbm.at[pl.ds(0, _T)],
                        ssem.at[1 - s]).wait()
                x_load(k + 1, 1 - s)
            else:
                pltpu.make_async_copy(
                    xb.at[1 - s], o_hbm.at[pl.ds(0, _T)],
                    ssem.at[1 - s]).wait()
                x_load(k + 1, 1 - s)

            xbs = xb.at[s]

            @plsc.parallel_loop(0, _T, step=16, unroll=8)
            def _(i):
                sl = pl.ds(i, 16)
                xbs[sl] = xbs[sl] + peb[sl]

            off = (k % Bn) * LD + base + (k // Bn) * _T
            pltpu.async_copy(xb.at[s], o_hbm.at[pl.ds(off, _T)], ssem.at[s])
        return 0

    lax.fori_loop(0, nt, t_body, 0)

    pltpu.make_async_copy(xb.at[0], o_hbm.at[pl.ds(0, _T)], ssem.at[0]).wait()
    pltpu.make_async_copy(xb.at[1], o_hbm.at[pl.ds(0, _T)], ssem.at[1]).wait()


def _sc_add(x1, pe1):
    return pl.kernel(
        _sc_body,
        out_type=jax.ShapeDtypeStruct(x1.shape, x1.dtype),
        mesh=plsc.VectorSubcoreMesh(core_axis_name="c", subcore_axis_name="s"),
        scratch_types=[
            pltpu.VMEM((_T,), jnp.float32),
            pltpu.VMEM((2, _T), jnp.float32),
            pltpu.SemaphoreType.DMA((2,)),
            pltpu.SemaphoreType.DMA((2,)),
        ],
    )(x1, pe1)


def kernel(x, pe):
    B, L, D = x.shape
    out1 = _sc_add(x.reshape(B * L * D), pe.reshape(L * D))
    return out1.reshape(B, L, D)


# speedup vs baseline: 4.6514x; 4.6514x over previous
import jax
import jax.numpy as jnp
from jax.experimental import pallas as pl


_TL = 512


def _add_kernel(x_ref, pe_ref, o_ref):
    o_ref[...] = x_ref[...] + pe_ref[...][None]


def kernel(x, pe):
    B, L, D = x.shape
    grid = (L // _TL, B)
    return pl.pallas_call(
        _add_kernel,
        grid=grid,
        in_specs=[
            pl.BlockSpec((1, _TL, D), lambda l, b: (b, l, 0)),
            pl.BlockSpec((_TL, D), lambda l, b: (l, 0)),
        ],
        out_specs=pl.BlockSpec((1, _TL, D), lambda l, b: (b, l, 0)),
        out_shape=jax.ShapeDtypeStruct((B, L, D), x.dtype),
    )(x, pe)
